# baseline (speedup 1.0000x reference)
import functools

import jax
import jax.numpy as jnp
from jax import lax
from jax.experimental import pallas as pl
from jax.experimental.pallas import tpu as pltpu
from jax.experimental.pallas import tpu_sc as plsc

B = 16384
D = 64
NUM_NEG = 4
NC = 2
NS = 16
NW = NC * NS
CHUNK = B // NW
NSEG = 2 + NUM_NEG


_mesh = plsc.VectorSubcoreMesh(core_axis_name="c", subcore_axis_name="s")


@functools.partial(
    pl.kernel,
    mesh=_mesh,
    out_type=jax.ShapeDtypeStruct((NSEG * B, D), jnp.float32),
    scratch_types=[
        pltpu.VMEM((CHUNK,), jnp.int32),
        pltpu.VMEM((CHUNK, D), jnp.float32),
        pltpu.SemaphoreType.DMA,
    ],
    compiler_params=pltpu.CompilerParams(use_tc_tiling_on_sc=False),
)
def _gather_kernel(user_w, item_w, u_ids, i_ids, ne_ids, out, idx_v, rows_v, sem):
    wid = lax.axis_index("s") * NC + lax.axis_index("c")
    base = wid * CHUNK
    for j in range(NSEG):
        if j == 0:
            ids, tab, src_off = u_ids, user_w, base
        elif j == 1:
            ids, tab, src_off = i_ids, item_w, base
        else:
            ids, tab, src_off = ne_ids, item_w, (j - 2) * B + base
        pltpu.sync_copy(ids.at[pl.ds(src_off, CHUNK)], idx_v)
        pltpu.async_copy(tab.at[idx_v], rows_v, sem).wait()
        pltpu.sync_copy(rows_v, out.at[pl.ds(j * B + base, CHUNK)])


def kernel(user_ids, item_ids, ne_item_ids, user_weight, item_weight):
    ne_flat = ne_item_ids.reshape(-1)
    return _gather_kernel(user_weight, item_weight, user_ids, item_ids, ne_flat)


# speedup vs baseline: 1.0033x; 1.0033x over previous
import functools

import jax
import jax.numpy as jnp
from jax import lax
from jax.experimental import pallas as pl
from jax.experimental.pallas import tpu as pltpu
from jax.experimental.pallas import tpu_sc as plsc

B = 16384
D = 64
NUM_NEG = 4
NC = 2
NS = 16
NW = NC * NS
CHUNK = B // NW
NSEG = 2 + NUM_NEG
NBUF = 3


_mesh = plsc.VectorSubcoreMesh(core_axis_name="c", subcore_axis_name="s")


@functools.partial(
    pl.kernel,
    mesh=_mesh,
    out_type=jax.ShapeDtypeStruct((NSEG * B, D), jnp.float32),
    scratch_types=(
        [pltpu.VMEM((CHUNK,), jnp.int32) for _ in range(NSEG)]
        + [pltpu.VMEM((CHUNK, D), jnp.float32) for _ in range(NBUF)]
        + [pltpu.SemaphoreType.DMA for _ in range(2 * NBUF + 1)]
    ),
    compiler_params=pltpu.CompilerParams(use_tc_tiling_on_sc=False),
)
def _gather_kernel(user_w, item_w, u_ids, i_ids, ne_ids, out,
                   ix0, ix1, ix2, ix3, ix4, ix5,
                   buf0, buf1, buf2, g0, g1, g2, s0, s1, s2, isem):
    idxs = (ix0, ix1, ix2, ix3, ix4, ix5)
    bufs = (buf0, buf1, buf2)
    gsem = (g0, g1, g2)
    ssem = (s0, s1, s2)
    wid = lax.axis_index("s") * NC + lax.axis_index("c")
    base = wid * CHUNK

    idx_copies = []
    for j in range(NSEG):
        if j == 0:
            src = u_ids.at[pl.ds(base, CHUNK)]
        elif j == 1:
            src = i_ids.at[pl.ds(base, CHUNK)]
        else:
            src = ne_ids.at[pl.ds((j - 2) * B + base, CHUNK)]
        idx_copies.append(pltpu.async_copy(src, idxs[j], isem))
    for c in idx_copies:
        c.wait()

    tabs = [user_w] + [item_w] * (NSEG - 1)
    gathers = [None] * NSEG
    stores = [None] * NSEG

    def start_gather(k):
        gathers[k] = pltpu.async_copy(
            tabs[k].at[idxs[k]], bufs[k % NBUF], gsem[k % NBUF])

    def start_store(k):
        stores[k] = pltpu.async_copy(
            bufs[k % NBUF], out.at[pl.ds(k * B + base, CHUNK)], ssem[k % NBUF])

    for k in range(NBUF):
        start_gather(k)
    for k in range(NSEG):
        gathers[k].wait()
        start_store(k)
        nk = k + NBUF
        if nk < NSEG:
            stores[nk - NBUF].wait()
            start_gather(nk)
    for k in range(NSEG - NBUF, NSEG):
        stores[k].wait()


def kernel(user_ids, item_ids, ne_item_ids, user_weight, item_weight):
    ne_flat = ne_item_ids.reshape(-1)
    return _gather_kernel(user_weight, item_weight, user_ids, item_ids, ne_flat)
